# baseline (speedup 1.0000x reference)
import functools

import jax
import jax.numpy as jnp
from jax import lax
from jax.experimental import pallas as pl
from jax.experimental.pallas import tpu as pltpu
from jax.experimental.pallas import tpu_sc as plsc

N = 10000
E = 320000
D = 128

NC = 2
NS = 16
NW = NC * NS
LANES = 16

GW = 40
CHUNKS = (38400, 121600, 121600, 38400)
NCHUNK = len(CHUNKS)
OFFS = tuple(sum(CHUNKS[:i]) for i in range(NCHUNK))
GS = 6
WS = 3
EBLK = 6400
NBLK = 2000


def _mesh():
    return plsc.VectorSubcoreMesh(core_axis_name="c", subcore_axis_name="s")


def _pq_body(nf, w1a, w1b, b1, p_out, q_out):
    x = nf[...]
    p_out[...] = jnp.dot(x, w1a[...], preferred_element_type=jnp.float32) + b1[...]
    q_out[...] = jnp.dot(x, w1b[...], preferred_element_type=jnp.float32)


def _compute_pq(nf, w1a, w1b, b1):
    blk = 2000
    return pl.pallas_call(
        _pq_body,
        grid=(N // blk,),
        in_specs=[
            pl.BlockSpec((blk, D), lambda i: (i, 0)),
            pl.BlockSpec((D, D), lambda i: (0, 0)),
            pl.BlockSpec((D, D), lambda i: (0, 0)),
            pl.BlockSpec((1, D), lambda i: (0, 0)),
        ],
        out_specs=[pl.BlockSpec((blk, D), lambda i: (i, 0)),
                   pl.BlockSpec((blk, D), lambda i: (i, 0))],
        out_shape=[jax.ShapeDtypeStruct((N, D), jnp.float32),
                   jax.ShapeDtypeStruct((N, D), jnp.float32)],
    )(nf, w1a, w1b, b1)


def _gather_add(p, q, senders, receivers, off, size):
    epw = size // NW
    nwin = epw // GW
    assert epw % GW == 0 and nwin >= 8

    @functools.partial(
        pl.kernel,
        mesh=_mesh(),
        out_type=jax.ShapeDtypeStruct((size, D), jnp.float32),
        scratch_types=[
            pltpu.VMEM((epw,), jnp.int32),
            pltpu.VMEM((epw,), jnp.int32),
            pltpu.VMEM((GS, GW, D), jnp.float32),
            pltpu.VMEM((GS, GW, D), jnp.float32),
            pltpu.VMEM((WS, GW, D), jnp.float32),
        ] + [pltpu.SemaphoreType.DMA] * (GS + WS),
    )
    def k(p_hbm, q_hbm, s_hbm, r_hbm, g_hbm, sidx_v, ridx_v, vi, vj, go,
          *sems):
        gsems = sems[:GS]
        wsems = sems[GS:]
        wid = lax.axis_index("s") * NC + lax.axis_index("c")
        base = wid * epw
        gbase = off + base
        pltpu.async_copy(s_hbm.at[pl.ds(gbase, epw)], sidx_v, gsems[0])
        pltpu.async_copy(r_hbm.at[pl.ds(gbase, epw)], ridx_v, gsems[1])
        pltpu.make_async_copy(s_hbm.at[pl.ds(gbase, epw)], sidx_v,
                              gsems[0]).wait()
        pltpu.make_async_copy(r_hbm.at[pl.ds(gbase, epw)], ridx_v,
                              gsems[1]).wait()

        def issue(w, b):
            off = w * GW
            pltpu.async_copy(p_hbm.at[sidx_v.at[pl.ds(off, GW)]], vi.at[b],
                             gsems[b])
            pltpu.async_copy(q_hbm.at[ridx_v.at[pl.ds(off, GW)]], vj.at[b],
                             gsems[b])

        def wait_gather(b):
            pltpu.make_async_copy(p_hbm.at[pl.ds(0, GW)], vi.at[b],
                                  gsems[b]).wait()
            pltpu.make_async_copy(p_hbm.at[pl.ds(0, GW)], vj.at[b],
                                  gsems[b]).wait()

        def wait_write(c):
            pltpu.make_async_copy(p_hbm.at[pl.ds(0, GW)], go.at[c],
                                  wsems[c]).wait()

        def vadd(b, c):
            @plsc.parallel_loop(0, GW, unroll=4)
            def _row(i):
                for ch in range(D // LANES):
                    sl = pl.ds(ch * LANES, LANES)
                    go[c, i, sl] = vi[b, i, sl] + vj[b, i, sl]

        def write(w, c):
            pltpu.async_copy(go.at[c], g_hbm.at[pl.ds(base + w * GW, GW)],
                             wsems[c])

        def step(w, b, c, first, last):
            wait_gather(b)
            if not first:
                wait_write(c)
            vadd(b, c)
            if not last:
                issue(w + GS, b)
            write(w, c)

        for w in range(GS):
            issue(w, w)
        for w in range(WS):
            step(w, w % GS, w % WS, True, w + GS > nwin - 1)
        p_extra = (nwin - GS - WS) % GS
        for w in range(WS, WS + p_extra):
            step(w, w % GS, w % WS, False, False)

        w0 = WS + p_extra
        m = (nwin - GS - WS) // GS
        gslots = tuple((w0 + t) % GS for t in range(GS))
        wslots = tuple((w0 + t) % WS for t in range(GS))

        @pl.loop(0, m)
        def _main(i):
            for t in range(GS):
                step(w0 + GS * i + t, gslots[t], wslots[t], False, False)

        for w in range(nwin - GS, nwin):
            step(w, w % GS, w % WS, False, True)
        for w in range(nwin - WS, nwin):
            wait_write(w % WS)

    return k(p, q, senders, receivers)


def _edge_body_noprev(g, ef, w1c, w2, b2, w3, b3, gam, bet, neo, ne):
    efb = ef[...]
    h = g[...] + jnp.dot(efb, w1c[...], preferred_element_type=jnp.float32)
    h = jnp.maximum(h, 0.0)
    h = jnp.dot(h, w2[...], preferred_element_type=jnp.float32) + b2[...]
    h = jnp.maximum(h, 0.0)
    h = jnp.dot(h, w3[...], preferred_element_type=jnp.float32) + b3[...]
    mu = jnp.mean(h, axis=-1, keepdims=True)
    hc = h - mu
    var = jnp.mean(hc * hc, axis=-1, keepdims=True)
    ln = gam[...] * hc / jnp.sqrt(var + 1e-5) + bet[...]
    ne[...] = ln
    neo[...] = ln + efb


def _edge_body_prev(g, ef, w1c, w2, b2, w3, b3, gam, bet, prev, neo, ne):
    del prev
    _edge_body_noprev(g, ef, w1c, w2, b2, w3, b3, gam, bet, neo, ne)


def _edge_mlp(g, ef, w1c, w2, b2, w3, b3, gam, bet, off_e, size, neo_prev):
    nblk = size // EBLK
    off = off_e // EBLK
    wspec = pl.BlockSpec((D, D), lambda i: (0, 0))
    bspec = pl.BlockSpec((1, D), lambda i: (0, 0))
    cspec = pl.BlockSpec((EBLK, D), lambda i: (i, 0))
    fspec = pl.BlockSpec((EBLK, D), lambda i: (i + off, 0))
    in_specs = [cspec, fspec, wspec, wspec, bspec, wspec, bspec, bspec,
                bspec]
    args = [g, ef, w1c, w2, b2, w3, b3, gam, bet]
    kwargs = {}
    if neo_prev is None:
        body = _edge_body_noprev
    else:
        body = _edge_body_prev
        in_specs = in_specs + [pl.BlockSpec(memory_space=pl.ANY)]
        args = args + [neo_prev]
        kwargs["input_output_aliases"] = {len(args) - 1: 0}
    return pl.pallas_call(
        body,
        grid=(nblk,),
        in_specs=in_specs,
        out_specs=[fspec, cspec],
        out_shape=[jax.ShapeDtypeStruct((E, D), jnp.float32),
                   jax.ShapeDtypeStruct((size, D), jnp.float32)],
        **kwargs,
    )(*args)


NSLOT = 6
LAG = NSLOT // 2


def _scatter_add(new_edge, receivers, zeros, off, size):
    epc = size // NC
    epw = epc // NS
    nwin = epw // GW
    assert epw % GW == 0 and nwin >= 8
    npw = 640

    @functools.partial(
        pl.kernel,
        mesh=_mesh(),
        out_type=jax.ShapeDtypeStruct((NC, N, D), jnp.float32),
        scratch_types=[
            pltpu.VMEM((NSLOT, GW), jnp.int32),
            pltpu.VMEM((NSLOT, GW, D), jnp.float32),
            pltpu.VMEM_SHARED((N, D), jnp.float32),
        ] + [pltpu.SemaphoreType.DMA] * (2 * NSLOT),
    )
    def k(e_hbm, r_hbm, z_hbm, out_hbm, idx_v, rows_v, acc, *sems):
        lsems = sems[:NSLOT]
        ssems = sems[NSLOT:]
        c = lax.axis_index("c")
        s = lax.axis_index("s")
        base = c * epc + s * epw
        rbase = off + base

        def issue_load(w, b):
            off = w * GW
            pltpu.async_copy(r_hbm.at[pl.ds(rbase + off, GW)], idx_v.at[b],
                             lsems[b])
            pltpu.async_copy(e_hbm.at[pl.ds(base + off, GW)], rows_v.at[b],
                             lsems[b])

        def wait_load(b):
            pltpu.make_async_copy(r_hbm.at[pl.ds(0, GW)], idx_v.at[b],
                                  lsems[b]).wait()
            pltpu.make_async_copy(e_hbm.at[pl.ds(0, GW)], rows_v.at[b],
                                  lsems[b]).wait()

        def scat(b):
            pltpu.async_copy(rows_v.at[b], acc.at[idx_v.at[b]], ssems[b],
                             add=True)

        def wait_scat(b):
            pltpu.make_async_copy(e_hbm.at[pl.ds(0, GW)], rows_v.at[b],
                                  ssems[b]).wait()

        for w in range(LAG):
            issue_load(w, w)
        row0 = jnp.minimum(s * npw, N - npw)
        pltpu.sync_copy(z_hbm.at[pl.ds(row0, npw)],
                        acc.at[pl.ds(row0, npw)])
        plsc.subcore_barrier()

        def step(w, b, first, last):
            wait_load(b)
            scat(b)
            if not last:
                b2 = (b + LAG) % NSLOT
                if not first:
                    wait_scat(b2)
                issue_load(w + LAG, b2)

        p_extra = (nwin - 2 * LAG) % NSLOT
        m = (nwin - 2 * LAG) // NSLOT
        for w in range(LAG):
            step(w, w, True, False)
        for w in range(LAG, LAG + p_extra):
            step(w, w % NSLOT, False, False)

        w0 = LAG + p_extra
        slots = tuple((w0 + t) % NSLOT for t in range(NSLOT))

        @pl.loop(0, m)
        def _main(i):
            for t in range(NSLOT):
                step(w0 + NSLOT * i + t, slots[t], False, False)

        for w in range(nwin - LAG, nwin):
            step(w, w % NSLOT, False, True)
        for b in range(NSLOT):
            wait_scat(b)

        plsc.subcore_barrier()
        pltpu.sync_copy(acc.at[pl.ds(row0, npw)],
                        out_hbm.at[c].at[pl.ds(row0, npw)])

    return k(new_edge, receivers, zeros)


def _node_body(*refs):
    nf = refs[0]
    parts = refs[1:1 + NCHUNK]
    (w1a, w1b, b1, w2, b2, w3, b3, gam, bet, out) = refs[1 + NCHUNK:]
    nfb = nf[...]
    ssum = parts[0][0] + parts[0][1]
    for pr in parts[1:]:
        ssum = ssum + pr[0] + pr[1]
    h = (jnp.dot(nfb, w1a[...], preferred_element_type=jnp.float32)
         + jnp.dot(ssum, w1b[...], preferred_element_type=jnp.float32)
         + b1[...])
    h = jnp.maximum(h, 0.0)
    h = jnp.dot(h, w2[...], preferred_element_type=jnp.float32) + b2[...]
    h = jnp.maximum(h, 0.0)
    h = jnp.dot(h, w3[...], preferred_element_type=jnp.float32) + b3[...]
    mu = jnp.mean(h, axis=-1, keepdims=True)
    hc = h - mu
    var = jnp.mean(hc * hc, axis=-1, keepdims=True)
    ln = gam[...] * hc / jnp.sqrt(var + 1e-5) + bet[...]
    out[...] = ln + nfb


def _node_mlp(nf, parts, w1a, w1b, b1, w2, b2, w3, b3, gam, bet):
    wspec = pl.BlockSpec((D, D), lambda i: (0, 0))
    bspec = pl.BlockSpec((1, D), lambda i: (0, 0))
    rspec = pl.BlockSpec((NBLK, D), lambda i: (i, 0))
    pspec = pl.BlockSpec((NC, NBLK, D), lambda i: (0, i, 0))
    return pl.pallas_call(
        _node_body,
        grid=(N // NBLK,),
        in_specs=[rspec] + [pspec] * NCHUNK + [wspec, wspec, bspec, wspec,
                                               bspec, wspec, bspec, bspec,
                                               bspec],
        out_specs=pl.BlockSpec((NBLK, D), lambda i: (i, 0)),
        out_shape=jax.ShapeDtypeStruct((N, D), jnp.float32),
    )(nf, *parts, w1a, w1b, b1, w2, b2, w3, b3, gam, bet)


def kernel(node_features, edge_features, senders, receivers,
           eW1, eb1, eW2, eb2, eW3, eb3, eg, ebt,
           nW1, nb1, nW2, nb2, nW3, nb3, ng, nbt):
    nf = node_features.reshape(N, D)
    ef = edge_features.reshape(E, D)

    p, q = _compute_pq(nf, eW1[:D], eW1[D:2 * D], eb1.reshape(1, D))
    zeros = jnp.zeros((N, D), jnp.float32)

    w1c = eW1[2 * D:]
    eb2r = eb2.reshape(1, D)
    eb3r = eb3.reshape(1, D)
    egr = eg.reshape(1, D)
    ebtr = ebt.reshape(1, D)

    neo = None
    parts = []
    for chunk in range(NCHUNK):
        off, size = OFFS[chunk], CHUNKS[chunk]
        g = _gather_add(p, q, senders, receivers, off, size)
        neo, ne = _edge_mlp(g, ef, w1c, eW2, eb2r, eW3, eb3r, egr, ebtr,
                            off, size, neo)
        parts.append(_scatter_add(ne, receivers, zeros, off, size))

    nn = _node_mlp(nf, parts, nW1[:D], nW1[D:2 * D], nb1.reshape(1, D),
                   nW2, nb2.reshape(1, D), nW3, nb3.reshape(1, D),
                   ng.reshape(1, D), nbt.reshape(1, D))
    return nn.reshape(1, N, D), neo.reshape(1, E, D)


# speedup vs baseline: 1.1015x; 1.1015x over previous
import functools

import jax
import jax.numpy as jnp
from jax import lax
from jax.experimental import pallas as pl
from jax.experimental.pallas import tpu as pltpu
from jax.experimental.pallas import tpu_sc as plsc

N = 10000
E = 320000
D = 128

NC = 2
NS = 16
NW = NC * NS
LANES = 16

GW = 40
CHUNKS = (160000, 160000)
NCHUNK = len(CHUNKS)
OFFS = tuple(sum(CHUNKS[:i]) for i in range(NCHUNK))
GS = 6
WS = 3
EBLK = 10000
NBLK = 2000


def _mesh():
    return plsc.VectorSubcoreMesh(core_axis_name="c", subcore_axis_name="s")


def _pq_body(nf, w1a, w1b, b1, p_out, q_out):
    x = nf[...]
    p_out[...] = jnp.dot(x, w1a[...], preferred_element_type=jnp.float32) + b1[...]
    q_out[...] = jnp.dot(x, w1b[...], preferred_element_type=jnp.float32)


def _compute_pq(nf, w1a, w1b, b1):
    blk = 2000
    return pl.pallas_call(
        _pq_body,
        grid=(N // blk,),
        in_specs=[
            pl.BlockSpec((blk, D), lambda i: (i, 0)),
            pl.BlockSpec((D, D), lambda i: (0, 0)),
            pl.BlockSpec((D, D), lambda i: (0, 0)),
            pl.BlockSpec((1, D), lambda i: (0, 0)),
        ],
        out_specs=[pl.BlockSpec((blk, D), lambda i: (i, 0)),
                   pl.BlockSpec((blk, D), lambda i: (i, 0))],
        out_shape=[jax.ShapeDtypeStruct((N, D), jnp.float32),
                   jax.ShapeDtypeStruct((N, D), jnp.float32)],
    )(nf, w1a, w1b, b1)


def _gather_add(p, q, senders, receivers, off, size):
    epw = size // NW
    nwin = epw // GW
    assert epw % GW == 0 and nwin >= 8

    @functools.partial(
        pl.kernel,
        mesh=_mesh(),
        out_type=jax.ShapeDtypeStruct((size, D), jnp.float32),
        scratch_types=[
            pltpu.VMEM((epw,), jnp.int32),
            pltpu.VMEM((epw,), jnp.int32),
            pltpu.VMEM((GS, GW, D), jnp.float32),
            pltpu.VMEM((GS, GW, D), jnp.float32),
            pltpu.VMEM((WS, GW, D), jnp.float32),
        ] + [pltpu.SemaphoreType.DMA] * (GS + WS),
    )
    def k(p_hbm, q_hbm, s_hbm, r_hbm, g_hbm, sidx_v, ridx_v, vi, vj, go,
          *sems):
        gsems = sems[:GS]
        wsems = sems[GS:]
        wid = lax.axis_index("s") * NC + lax.axis_index("c")
        base = wid * epw
        gbase = off + base
        pltpu.async_copy(s_hbm.at[pl.ds(gbase, epw)], sidx_v, gsems[0])
        pltpu.async_copy(r_hbm.at[pl.ds(gbase, epw)], ridx_v, gsems[1])
        pltpu.make_async_copy(s_hbm.at[pl.ds(gbase, epw)], sidx_v,
                              gsems[0]).wait()
        pltpu.make_async_copy(r_hbm.at[pl.ds(gbase, epw)], ridx_v,
                              gsems[1]).wait()

        def issue(w, b):
            off = w * GW
            pltpu.async_copy(p_hbm.at[sidx_v.at[pl.ds(off, GW)]], vi.at[b],
                             gsems[b])
            pltpu.async_copy(q_hbm.at[ridx_v.at[pl.ds(off, GW)]], vj.at[b],
                             gsems[b])

        def wait_gather(b):
            pltpu.make_async_copy(p_hbm.at[pl.ds(0, GW)], vi.at[b],
                                  gsems[b]).wait()
            pltpu.make_async_copy(p_hbm.at[pl.ds(0, GW)], vj.at[b],
                                  gsems[b]).wait()

        def wait_write(c):
            pltpu.make_async_copy(p_hbm.at[pl.ds(0, GW)], go.at[c],
                                  wsems[c]).wait()

        def vadd(b, c):
            @plsc.parallel_loop(0, GW, unroll=4)
            def _row(i):
                for ch in range(D // LANES):
                    sl = pl.ds(ch * LANES, LANES)
                    go[c, i, sl] = vi[b, i, sl] + vj[b, i, sl]

        def write(w, c):
            pltpu.async_copy(go.at[c], g_hbm.at[pl.ds(base + w * GW, GW)],
                             wsems[c])

        def step(w, b, c, first, last):
            wait_gather(b)
            if not first:
                wait_write(c)
            vadd(b, c)
            if not last:
                issue(w + GS, b)
            write(w, c)

        for w in range(GS):
            issue(w, w)
        for w in range(WS):
            step(w, w % GS, w % WS, True, w + GS > nwin - 1)
        p_extra = (nwin - GS - WS) % GS
        for w in range(WS, WS + p_extra):
            step(w, w % GS, w % WS, False, False)

        w0 = WS + p_extra
        m = (nwin - GS - WS) // GS
        gslots = tuple((w0 + t) % GS for t in range(GS))
        wslots = tuple((w0 + t) % WS for t in range(GS))

        @pl.loop(0, m)
        def _main(i):
            for t in range(GS):
                step(w0 + GS * i + t, gslots[t], wslots[t], False, False)

        for w in range(nwin - GS, nwin):
            step(w, w % GS, w % WS, False, True)
        for w in range(nwin - WS, nwin):
            wait_write(w % WS)

    return k(p, q, senders, receivers)


def _edge_body_noprev(g, ef, w1c, w2, b2, w3, b3, gam, bet, neo, ne):
    efb = ef[...]
    h = g[...] + jnp.dot(efb, w1c[...], preferred_element_type=jnp.float32)
    h = jnp.maximum(h, 0.0)
    h = jnp.dot(h, w2[...], preferred_element_type=jnp.float32) + b2[...]
    h = jnp.maximum(h, 0.0)
    h = jnp.dot(h, w3[...], preferred_element_type=jnp.float32) + b3[...]
    mu = jnp.mean(h, axis=-1, keepdims=True)
    hc = h - mu
    var = jnp.mean(hc * hc, axis=-1, keepdims=True)
    ln = gam[...] * hc / jnp.sqrt(var + 1e-5) + bet[...]
    ne[...] = ln
    neo[...] = ln + efb


def _edge_body_prev(g, ef, w1c, w2, b2, w3, b3, gam, bet, prev, neo, ne):
    del prev
    _edge_body_noprev(g, ef, w1c, w2, b2, w3, b3, gam, bet, neo, ne)


def _edge_mlp(g, ef, w1c, w2, b2, w3, b3, gam, bet, off_e, size, neo_prev):
    nblk = size // EBLK
    off = off_e // EBLK
    wspec = pl.BlockSpec((D, D), lambda i: (0, 0))
    bspec = pl.BlockSpec((1, D), lambda i: (0, 0))
    cspec = pl.BlockSpec((EBLK, D), lambda i: (i, 0))
    fspec = pl.BlockSpec((EBLK, D), lambda i: (i + off, 0))
    in_specs = [cspec, fspec, wspec, wspec, bspec, wspec, bspec, bspec,
                bspec]
    args = [g, ef, w1c, w2, b2, w3, b3, gam, bet]
    kwargs = {}
    if neo_prev is None:
        body = _edge_body_noprev
    else:
        body = _edge_body_prev
        in_specs = in_specs + [pl.BlockSpec(memory_space=pl.ANY)]
        args = args + [neo_prev]
        kwargs["input_output_aliases"] = {len(args) - 1: 0}
    return pl.pallas_call(
        body,
        grid=(nblk,),
        in_specs=in_specs,
        out_specs=[fspec, cspec],
        out_shape=[jax.ShapeDtypeStruct((E, D), jnp.float32),
                   jax.ShapeDtypeStruct((size, D), jnp.float32)],
        **kwargs,
    )(*args)


NSLOT = 6
LAG = NSLOT // 2


def _scatter_add(new_edge, receivers, zeros, off, size):
    epc = size // NC
    epw = epc // NS
    nwin = epw // GW
    assert epw % GW == 0 and nwin >= 8
    npw = 640

    @functools.partial(
        pl.kernel,
        mesh=_mesh(),
        out_type=jax.ShapeDtypeStruct((NC, N, D), jnp.float32),
        scratch_types=[
            pltpu.VMEM((NSLOT, GW), jnp.int32),
            pltpu.VMEM((NSLOT, GW, D), jnp.float32),
            pltpu.VMEM_SHARED((N, D), jnp.float32),
        ] + [pltpu.SemaphoreType.DMA] * (2 * NSLOT),
    )
    def k(e_hbm, r_hbm, z_hbm, out_hbm, idx_v, rows_v, acc, *sems):
        lsems = sems[:NSLOT]
        ssems = sems[NSLOT:]
        c = lax.axis_index("c")
        s = lax.axis_index("s")
        base = c * epc + s * epw
        rbase = off + base

        def issue_load(w, b):
            off = w * GW
            pltpu.async_copy(r_hbm.at[pl.ds(rbase + off, GW)], idx_v.at[b],
                             lsems[b])
            pltpu.async_copy(e_hbm.at[pl.ds(base + off, GW)], rows_v.at[b],
                             lsems[b])

        def wait_load(b):
            pltpu.make_async_copy(r_hbm.at[pl.ds(0, GW)], idx_v.at[b],
                                  lsems[b]).wait()
            pltpu.make_async_copy(e_hbm.at[pl.ds(0, GW)], rows_v.at[b],
                                  lsems[b]).wait()

        def scat(b):
            pltpu.async_copy(rows_v.at[b], acc.at[idx_v.at[b]], ssems[b],
                             add=True)

        def wait_scat(b):
            pltpu.make_async_copy(e_hbm.at[pl.ds(0, GW)], rows_v.at[b],
                                  ssems[b]).wait()

        for w in range(LAG):
            issue_load(w, w)
        row0 = jnp.minimum(s * npw, N - npw)
        pltpu.sync_copy(z_hbm.at[pl.ds(row0, npw)],
                        acc.at[pl.ds(row0, npw)])
        plsc.subcore_barrier()

        def step(w, b, first, last):
            wait_load(b)
            scat(b)
            if not last:
                b2 = (b + LAG) % NSLOT
                if not first:
                    wait_scat(b2)
                issue_load(w + LAG, b2)

        p_extra = (nwin - 2 * LAG) % NSLOT
        m = (nwin - 2 * LAG) // NSLOT
        for w in range(LAG):
            step(w, w, True, False)
        for w in range(LAG, LAG + p_extra):
            step(w, w % NSLOT, False, False)

        w0 = LAG + p_extra
        slots = tuple((w0 + t) % NSLOT for t in range(NSLOT))

        @pl.loop(0, m)
        def _main(i):
            for t in range(NSLOT):
                step(w0 + NSLOT * i + t, slots[t], False, False)

        for w in range(nwin - LAG, nwin):
            step(w, w % NSLOT, False, True)
        for b in range(NSLOT):
            wait_scat(b)

        plsc.subcore_barrier()
        pltpu.sync_copy(acc.at[pl.ds(row0, npw)],
                        out_hbm.at[c].at[pl.ds(row0, npw)])

    return k(new_edge, receivers, zeros)


def _node_body(*refs):
    nf = refs[0]
    parts = refs[1:1 + NCHUNK]
    (w1a, w1b, b1, w2, b2, w3, b3, gam, bet, out) = refs[1 + NCHUNK:]
    nfb = nf[...]
    ssum = parts[0][0] + parts[0][1]
    for pr in parts[1:]:
        ssum = ssum + pr[0] + pr[1]
    h = (jnp.dot(nfb, w1a[...], preferred_element_type=jnp.float32)
         + jnp.dot(ssum, w1b[...], preferred_element_type=jnp.float32)
         + b1[...])
    h = jnp.maximum(h, 0.0)
    h = jnp.dot(h, w2[...], preferred_element_type=jnp.float32) + b2[...]
    h = jnp.maximum(h, 0.0)
    h = jnp.dot(h, w3[...], preferred_element_type=jnp.float32) + b3[...]
    mu = jnp.mean(h, axis=-1, keepdims=True)
    hc = h - mu
    var = jnp.mean(hc * hc, axis=-1, keepdims=True)
    ln = gam[...] * hc / jnp.sqrt(var + 1e-5) + bet[...]
    out[...] = ln + nfb


def _node_mlp(nf, parts, w1a, w1b, b1, w2, b2, w3, b3, gam, bet):
    wspec = pl.BlockSpec((D, D), lambda i: (0, 0))
    bspec = pl.BlockSpec((1, D), lambda i: (0, 0))
    rspec = pl.BlockSpec((NBLK, D), lambda i: (i, 0))
    pspec = pl.BlockSpec((NC, NBLK, D), lambda i: (0, i, 0))
    return pl.pallas_call(
        _node_body,
        grid=(N // NBLK,),
        in_specs=[rspec] + [pspec] * NCHUNK + [wspec, wspec, bspec, wspec,
                                               bspec, wspec, bspec, bspec,
                                               bspec],
        out_specs=pl.BlockSpec((NBLK, D), lambda i: (i, 0)),
        out_shape=jax.ShapeDtypeStruct((N, D), jnp.float32),
    )(nf, *parts, w1a, w1b, b1, w2, b2, w3, b3, gam, bet)


def kernel(node_features, edge_features, senders, receivers,
           eW1, eb1, eW2, eb2, eW3, eb3, eg, ebt,
           nW1, nb1, nW2, nb2, nW3, nb3, ng, nbt):
    nf = node_features.reshape(N, D)
    ef = edge_features.reshape(E, D)

    p, q = _compute_pq(nf, eW1[:D], eW1[D:2 * D], eb1.reshape(1, D))
    zeros = jnp.zeros((N, D), jnp.float32)

    w1c = eW1[2 * D:]
    eb2r = eb2.reshape(1, D)
    eb3r = eb3.reshape(1, D)
    egr = eg.reshape(1, D)
    ebtr = ebt.reshape(1, D)

    neo = None
    parts = []
    for chunk in range(NCHUNK):
        off, size = OFFS[chunk], CHUNKS[chunk]
        g = _gather_add(p, q, senders, receivers, off, size)
        neo, ne = _edge_mlp(g, ef, w1c, eW2, eb2r, eW3, eb3r, egr, ebtr,
                            off, size, neo)
        parts.append(_scatter_add(ne, receivers, zeros, off, size))

    nn = _node_mlp(nf, parts, nW1[:D], nW1[D:2 * D], nb1.reshape(1, D),
                   nW2, nb2.reshape(1, D), nW3, nb3.reshape(1, D),
                   ng.reshape(1, D), nbt.reshape(1, D))
    return nn.reshape(1, N, D), neo.reshape(1, E, D)
